# Initial kernel scaffold; baseline (speedup 1.0000x reference)
import jax
import jax.numpy as jnp
from jax.experimental import pallas as pl


def kernel(x, emb_table):
    raise NotImplementedError("write your pallas kernel here")


# speedup vs baseline: 2.2772x; 2.2772x over previous
import jax
import jax.numpy as jnp
from jax.experimental import pallas as pl

MAX_LEN = 4096
BATCH = 2
D_MODEL = 1024

BLOCK_P = 512


def _add_body(x_ref, e_ref, o_ref):
    o_ref[...] = x_ref[...] + e_ref[...][:, None, :]


def kernel(x, emb_table):
    grid = (MAX_LEN // BLOCK_P,)
    return pl.pallas_call(
        _add_body,
        grid=grid,
        in_specs=[
            pl.BlockSpec((BLOCK_P, BATCH, D_MODEL), lambda i: (i, 0, 0)),
            pl.BlockSpec((BLOCK_P, D_MODEL), lambda i: (i, 0)),
        ],
        out_specs=pl.BlockSpec((BLOCK_P, BATCH, D_MODEL), lambda i: (i, 0, 0)),
        out_shape=jax.ShapeDtypeStruct((MAX_LEN, BATCH, D_MODEL), jnp.float32),
    )(x, emb_table)
